# baseline (speedup 1.0000x reference)
import functools

import jax
import jax.numpy as jnp
from jax import lax
from jax.experimental import pallas as pl
from jax.experimental.pallas import tpu as pltpu
from jax.experimental.pallas import tpu_sc as plsc

EMBED_DIM = 32
BATCH = 16384
HIST = 20
VOCAB = 1000000

PACK = 4
LINE_W = 128
ROWS_PER_STEP = 16384
N_STEPS = 16
N_LINES = N_STEPS * ROWS_PER_STEP
IN_BLOCKS = -(-VOCAB // ROWS_PER_STEP)

NUM_CORES = 2
NUM_SUBCORES = 16
NUM_WORKERS = NUM_CORES * NUM_SUBCORES
SPW = BATCH // NUM_WORKERS
CHUNK = 64
NUM_CHUNKS = SPW // CHUNK
ROWS_PER_CHUNK = CHUNK * HIST
IDX_W = 128
DMAS_PER_CHUNK = ROWS_PER_CHUNK // IDX_W
IDX_ROWS = SPW * HIST // IDX_W
ACC_ROWS = NUM_SUBCORES * SPW + NUM_SUBCORES
ZCHUNK = 64


def _repack_body(t0_ref, t1_ref, t2_ref, t3_ref, out_ref):
    for r, ref in enumerate((t0_ref, t1_ref, t2_ref, t3_ref)):
        out_ref[:, r * EMBED_DIM:(r + 1) * EMBED_DIM] = ref[...].T


def _sc_body(table_hbm, x_hbm, dst_hbm, out_hbm, idx_v, dst_v, rows_v,
             zeros_v, acc_s, gsem, ssem):
    sid = lax.axis_index("s")
    cid = lax.axis_index("c")
    wid = sid * NUM_CORES + cid
    wbase = pl.multiple_of(wid * SPW, SPW)
    xrow = pl.multiple_of(wid * IDX_ROWS, IDX_ROWS)
    arow = pl.multiple_of(sid * SPW, SPW)

    pltpu.sync_copy(x_hbm.at[pl.ds(xrow, IDX_ROWS)], idx_v)
    pltpu.sync_copy(dst_hbm.at[pl.ds(xrow, IDX_ROWS)], dst_v)

    zero = jnp.zeros((16,), jnp.float32)
    for i in range(ZCHUNK):
        zeros_v[i, pl.ds(0, 16)] = zero
        zeros_v[i, pl.ds(16, 16)] = zero
    for z in range(SPW // ZCHUNK):
        pltpu.sync_copy(zeros_v, acc_s.at[pl.ds(arow + z * ZCHUNK, ZCHUNK)])

    def fire_gathers(c):
        buf = (c % 2) * ROWS_PER_CHUNK
        return [
            pltpu.async_copy(
                table_hbm.at[idx_v.at[c * DMAS_PER_CHUNK + j]],
                rows_v.at[pl.ds(buf + j * IDX_W, IDX_W)],
                gsem,
            )
            for j in range(DMAS_PER_CHUNK)
        ]

    def fire_scatters(c):
        buf = (c % 2) * ROWS_PER_CHUNK
        return [
            pltpu.async_copy(
                rows_v.at[pl.ds(buf + j * IDX_W, IDX_W)],
                acc_s.at[dst_v.at[c * DMAS_PER_CHUNK + j]],
                ssem,
                add=True,
            )
            for j in range(DMAS_PER_CHUNK)
        ]

    gathers = fire_gathers(0)
    scatters = []
    for c in range(NUM_CHUNKS):
        for cp in gathers:
            cp.wait()
        new_scatters = fire_scatters(c)
        if c + 1 < NUM_CHUNKS:
            for cp in scatters:
                cp.wait()
            gathers = fire_gathers(c + 1)
        else:
            for cp in scatters:
                cp.wait()
        scatters = new_scatters
    for cp in scatters:
        cp.wait()

    pltpu.sync_copy(acc_s.at[pl.ds(arow, SPW)],
                    out_hbm.at[pl.ds(wbase, SPW)])


def _scale_body(sums_ref, lens_ref, out_ref):
    lens = lens_ref[...].astype(jnp.float32)
    inv = 1.0 / jnp.maximum(lens, 1.0)
    out_ref[...] = sums_ref[...] * inv


@jax.jit
def _run(tt, x2d, dst2d, lens):
    t128 = pl.pallas_call(
        _repack_body,
        grid=(N_STEPS,),
        in_specs=[
            pl.BlockSpec(
                (EMBED_DIM, ROWS_PER_STEP),
                functools.partial(
                    lambda i, r: (0, jnp.minimum(i + r * N_STEPS,
                                                 IN_BLOCKS - 1)), r=r))
            for r in range(PACK)
        ],
        out_specs=pl.BlockSpec((ROWS_PER_STEP, LINE_W), lambda i: (i, 0)),
        out_shape=jax.ShapeDtypeStruct((N_LINES, LINE_W), jnp.float32),
    )(tt, tt, tt, tt)
    t32 = t128.reshape(N_LINES * PACK, EMBED_DIM)

    mesh = plsc.VectorSubcoreMesh(core_axis_name="c", subcore_axis_name="s")
    sums = functools.partial(
        pl.kernel,
        mesh=mesh,
        out_type=jax.ShapeDtypeStruct((BATCH, EMBED_DIM), jnp.float32),
        scratch_types=[
            pltpu.VMEM((IDX_ROWS, IDX_W), jnp.int32),
            pltpu.VMEM((IDX_ROWS, IDX_W), jnp.int32),
            pltpu.VMEM((2 * ROWS_PER_CHUNK, EMBED_DIM), jnp.float32),
            pltpu.VMEM((ZCHUNK, EMBED_DIM), jnp.float32),
            pltpu.VMEM_SHARED((ACC_ROWS, EMBED_DIM), jnp.float32),
            pltpu.SemaphoreType.DMA,
            pltpu.SemaphoreType.DMA,
        ],
        compiler_params=pltpu.CompilerParams(use_tc_tiling_on_sc=False),
    )(_sc_body)(t32, x2d, dst2d)

    return pl.pallas_call(
        _scale_body,
        out_shape=jax.ShapeDtypeStruct((BATCH, EMBED_DIM), jnp.float32),
    )(sums, lens.reshape(BATCH, 1))


def kernel(x, sequence_lengths, table):
    lens = sequence_lengths.astype(jnp.int32)
    xi = x.astype(jnp.int32)
    b = jnp.arange(BATCH, dtype=jnp.int32)
    slot = ((b // SPW) // NUM_CORES) * SPW + b % SPW
    trash = NUM_SUBCORES * SPW + (b // SPW) // NUM_CORES
    t = jnp.arange(HIST, dtype=jnp.int32)[None, :]
    valid = t < lens[:, None]
    dst = jnp.where(valid, slot[:, None], trash[:, None])
    vmap = PACK * (xi % N_LINES) + xi // N_LINES
    x2d = vmap.reshape(BATCH * HIST // IDX_W, IDX_W)
    dst2d = dst.reshape(BATCH * HIST // IDX_W, IDX_W)
    return _run(table.T, x2d, dst2d, lens)


# speedup vs baseline: 1.0550x; 1.0550x over previous
import functools

import jax
import jax.numpy as jnp
from jax import lax
from jax.experimental import pallas as pl
from jax.experimental.pallas import tpu as pltpu
from jax.experimental.pallas import tpu_sc as plsc

EMBED_DIM = 32
BATCH = 16384
HIST = 20
VOCAB = 1000000

PACK = 4
LINE_W = 128
ROWS_PER_STEP = 16384
N_STEPS = 16
N_LINES = N_STEPS * ROWS_PER_STEP
IN_BLOCKS = -(-VOCAB // ROWS_PER_STEP)

NUM_CORES = 2
NUM_SUBCORES = 16
NUM_WORKERS = NUM_CORES * NUM_SUBCORES
SPW = BATCH // NUM_WORKERS
CHUNK = 64
NUM_CHUNKS = SPW // CHUNK
ROWS_PER_CHUNK = CHUNK * HIST
IDX_W = 128
DMAS_PER_CHUNK = ROWS_PER_CHUNK // IDX_W
IDX_ROWS = SPW * HIST // IDX_W
ACC_ROWS = NUM_SUBCORES * SPW + NUM_SUBCORES
ZCHUNK = 64


def _repack_body(t0_ref, t1_ref, t2_ref, t3_ref, out_ref):
    for r, ref in enumerate((t0_ref, t1_ref, t2_ref, t3_ref)):
        out_ref[:, r * EMBED_DIM:(r + 1) * EMBED_DIM] = ref[...].T


def _sc_body(table_hbm, xd_hbm, out_hbm, idx_v, dst_v, rows_v,
             zeros_v, acc_s, gsem, ssem):
    sid = lax.axis_index("s")
    cid = lax.axis_index("c")
    wid = sid * NUM_CORES + cid
    wbase = pl.multiple_of(wid * SPW, SPW)
    xrow = pl.multiple_of(wid * IDX_ROWS, IDX_ROWS)
    arow = pl.multiple_of(sid * SPW, SPW)

    pltpu.sync_copy(xd_hbm.at[pl.ds(xrow, IDX_ROWS)], idx_v)

    trash_row = NUM_SUBCORES * SPW + sid

    def unpack_body(j, _):
        for h in range(IDX_W // 16):
            w = idx_v[j, pl.ds(h * 16, 16)]
            rel = lax.shift_right_logical(w, 20)
            dst = jnp.where(rel >= SPW, trash_row, arow + rel)
            dst_v[j, pl.ds(h * 16, 16)] = dst
            idx_v[j, pl.ds(h * 16, 16)] = lax.bitwise_and(w, (1 << 20) - 1)
        return 0

    lax.fori_loop(0, IDX_ROWS, unpack_body, 0)

    zero = jnp.zeros((16,), jnp.float32)
    for i in range(ZCHUNK):
        zeros_v[i, pl.ds(0, 16)] = zero
        zeros_v[i, pl.ds(16, 16)] = zero
    for z in range(SPW // ZCHUNK):
        pltpu.sync_copy(zeros_v, acc_s.at[pl.ds(arow + z * ZCHUNK, ZCHUNK)])

    def fire_gathers(c):
        buf = (c % 2) * ROWS_PER_CHUNK
        return [
            pltpu.async_copy(
                table_hbm.at[idx_v.at[c * DMAS_PER_CHUNK + j]],
                rows_v.at[pl.ds(buf + j * IDX_W, IDX_W)],
                gsem,
            )
            for j in range(DMAS_PER_CHUNK)
        ]

    def fire_scatters(c):
        buf = (c % 2) * ROWS_PER_CHUNK
        return [
            pltpu.async_copy(
                rows_v.at[pl.ds(buf + j * IDX_W, IDX_W)],
                acc_s.at[dst_v.at[c * DMAS_PER_CHUNK + j]],
                ssem,
                add=True,
            )
            for j in range(DMAS_PER_CHUNK)
        ]

    gathers = fire_gathers(0)
    scatters = []
    for c in range(NUM_CHUNKS):
        for cp in gathers:
            cp.wait()
        new_scatters = fire_scatters(c)
        if c + 1 < NUM_CHUNKS:
            for cp in scatters:
                cp.wait()
            gathers = fire_gathers(c + 1)
        else:
            for cp in scatters:
                cp.wait()
        scatters = new_scatters
    for cp in scatters:
        cp.wait()

    pltpu.sync_copy(acc_s.at[pl.ds(arow, SPW)],
                    out_hbm.at[pl.ds(wbase, SPW)])


def _scale_body(sums_ref, lens_ref, out_ref):
    lens = lens_ref[...].astype(jnp.float32)
    inv = 1.0 / jnp.maximum(lens, 1.0)
    out_ref[...] = sums_ref[...] * inv


@jax.jit
def _run(tt, xd2d, lens):
    t128 = pl.pallas_call(
        _repack_body,
        grid=(N_STEPS,),
        in_specs=[
            pl.BlockSpec(
                (EMBED_DIM, ROWS_PER_STEP),
                functools.partial(
                    lambda i, r: (0, jnp.minimum(i + r * N_STEPS,
                                                 IN_BLOCKS - 1)), r=r))
            for r in range(PACK)
        ],
        out_specs=pl.BlockSpec((ROWS_PER_STEP, LINE_W), lambda i: (i, 0)),
        out_shape=jax.ShapeDtypeStruct((N_LINES, LINE_W), jnp.float32),
    )(tt, tt, tt, tt)
    t32 = t128.reshape(N_LINES * PACK, EMBED_DIM)

    mesh = plsc.VectorSubcoreMesh(core_axis_name="c", subcore_axis_name="s")
    sums = functools.partial(
        pl.kernel,
        mesh=mesh,
        out_type=jax.ShapeDtypeStruct((BATCH, EMBED_DIM), jnp.float32),
        scratch_types=[
            pltpu.VMEM((IDX_ROWS, IDX_W), jnp.int32),
            pltpu.VMEM((IDX_ROWS, IDX_W), jnp.int32),
            pltpu.VMEM((2 * ROWS_PER_CHUNK, EMBED_DIM), jnp.float32),
            pltpu.VMEM((ZCHUNK, EMBED_DIM), jnp.float32),
            pltpu.VMEM_SHARED((ACC_ROWS, EMBED_DIM), jnp.float32),
            pltpu.SemaphoreType.DMA,
            pltpu.SemaphoreType.DMA,
        ],
        compiler_params=pltpu.CompilerParams(use_tc_tiling_on_sc=False),
    )(_sc_body)(t32, xd2d)

    return pl.pallas_call(
        _scale_body,
        out_shape=jax.ShapeDtypeStruct((BATCH, EMBED_DIM), jnp.float32),
    )(sums, lens.reshape(BATCH, 1))


def kernel(x, sequence_lengths, table):
    lens = sequence_lengths.astype(jnp.int32)
    xi = x.astype(jnp.int32)
    b = jnp.arange(BATCH, dtype=jnp.int32)
    t = jnp.arange(HIST, dtype=jnp.int32)[None, :]
    valid = t < lens[:, None]
    dst = jnp.where(valid, (b % SPW)[:, None], SPW)
    vmap = PACK * (xi % N_LINES) + xi // N_LINES
    xd2d = (vmap | (dst << 20)).reshape(BATCH * HIST // IDX_W, IDX_W)
    return _run(table.T, xd2d, lens)
